# baseline (speedup 1.0000x reference)
import functools

import jax
import jax.numpy as jnp
from jax import lax
from jax.experimental import pallas as pl
from jax.experimental.pallas import tpu as pltpu
from jax.experimental.pallas import tpu_sc as plsc

N_NODES = 10000
N_EDGES = 320000
IN_DIM = 128
HID = 256
Z_DIM = 128

NPAD = 10240
NCORE = 2
NSUB = 16
NW = NCORE * NSUB
CH = 128
IDX_BITS = 14
NCHUNK = (N_EDGES + NW * CH - 1) // (NW * CH)
EPAD = NW * NCHUNK * CH

ROWS_PER_SUB = NPAD // NSUB
BLK = 512
GRID = NPAD // BLK

_mesh = plsc.VectorSubcoreMesh(core_axis_name="c", subcore_axis_name="s")


@functools.partial(
    pl.kernel,
    out_type=jax.ShapeDtypeStruct((NCORE, NPAD), jnp.float32),
    mesh=_mesh,
    scratch_types=[
        pltpu.VMEM((NCHUNK * CH,), jnp.int32),
        pltpu.VMEM((NCHUNK * CH,), jnp.float32),
        pltpu.VMEM_SHARED((NPAD,), jnp.float32),
    ],
)
def _deg_kernel(dst_hbm, ones_hbm, zeros1_hbm, degp_hbm, dst_v, ones_v, deg_sh):
    c = lax.axis_index("c")
    s = lax.axis_index("s")
    wid = c * NSUB + s
    pltpu.sync_copy(dst_hbm.at[wid], dst_v)
    pltpu.sync_copy(ones_hbm, ones_v)
    pltpu.sync_copy(zeros1_hbm.at[pl.ds(s * ROWS_PER_SUB, ROWS_PER_SUB)],
                    deg_sh.at[pl.ds(s * ROWS_PER_SUB, ROWS_PER_SUB)])
    plsc.subcore_barrier()

    pltpu.sync_copy(ones_v, deg_sh.at[dst_v], add=True)
    plsc.subcore_barrier()

    @pl.when(s == 0)
    def _():
        pltpu.sync_copy(deg_sh, degp_hbm.at[c])


@functools.partial(
    pl.kernel,
    out_type=jax.ShapeDtypeStruct((NCORE, NPAD, 128), jnp.float32),
    mesh=_mesh,
    scratch_types=[
        pltpu.VMEM((NCHUNK, CH), jnp.int32),
        pltpu.VMEM((2, CH), jnp.int32),
        pltpu.VMEM((2, CH), jnp.int32),
        pltpu.VMEM((CH, 128), jnp.float32),
        pltpu.VMEM((CH, 128), jnp.float32),
        pltpu.VMEM_SHARED((NPAD, 128), jnp.float32),
        pltpu.SemaphoreType.DMA,
        pltpu.SemaphoreType.DMA,
    ],
)
def _agg_kernel(xp_hbm, packed_hbm, zeros2_hbm, out_hbm,
                packed_v, sidx, didx, rows0, rows1, acc_sh, sem0, sem1):
    c = lax.axis_index("c")
    s = lax.axis_index("s")
    wid = c * NSUB + s
    pltpu.sync_copy(packed_hbm.at[wid], packed_v)
    pltpu.sync_copy(zeros2_hbm.at[pl.ds(s * ROWS_PER_SUB, ROWS_PER_SUB)],
                    acc_sh.at[pl.ds(s * ROWS_PER_SUB, ROWS_PER_SUB)])
    plsc.subcore_barrier()

    def unpack(j, b):
        for k in range(CH // 16):
            v = packed_v[j, pl.ds(k * 16, 16)]
            sidx[b, pl.ds(k * 16, 16)] = lax.shift_right_logical(v, IDX_BITS)
            didx[b, pl.ds(k * 16, 16)] = lax.bitwise_and(v, (1 << IDX_BITS) - 1)

    bufs = (rows0, rows1)
    sems = (sem0, sem1)
    unpack(0, 0)
    pltpu.async_copy(xp_hbm.at[sidx.at[0]], rows0, sem0)
    unpack(1, 1)
    pltpu.async_copy(xp_hbm.at[sidx.at[1]], rows1, sem1)

    def body(g, carry):
        for b in range(2):
            j = g * 2 + b
            pltpu.make_async_copy(xp_hbm.at[pl.ds(0, CH)], bufs[b],
                                  sems[b]).wait()
            pltpu.sync_copy(bufs[b], acc_sh.at[didx.at[b]], add=True)

            @pl.when(j + 2 < NCHUNK)
            def _():
                unpack(j + 2, b)
                pltpu.async_copy(xp_hbm.at[sidx.at[b]], bufs[b], sems[b])

        return carry

    lax.fori_loop(0, NCHUNK // 2, body, 0)
    last = NCHUNK - 1
    if NCHUNK % 2:
        pltpu.make_async_copy(xp_hbm.at[pl.ds(0, CH)], bufs[last % 2],
                              sems[last % 2]).wait()
        pltpu.sync_copy(bufs[last % 2], acc_sh.at[didx.at[last % 2]], add=True)
    plsc.subcore_barrier()
    pltpu.sync_copy(acc_sh.at[pl.ds(s * ROWS_PER_SUB, ROWS_PER_SUB)],
                    out_hbm.at[c, pl.ds(s * ROWS_PER_SUB, ROWS_PER_SUB)])


def _dinv_of(dp_ref):
    return lax.rsqrt(dp_ref[:, 0:1] + dp_ref[:, 1:2] + 1.0)


def _scale_body(dp_ref, x_ref, o_ref):
    o_ref[...] = x_ref[...] * _dinv_of(dp_ref)


def _mm_body(dp_ref, p0_ref, p1_ref, xp_ref, w1_ref, b1_ref, w2_ref, o_ref):
    dinv = _dinv_of(dp_ref)
    agg1 = (p0_ref[...] + p1_ref[...] + xp_ref[...]) * dinv
    h = jnp.maximum(
        jnp.dot(agg1.astype(jnp.bfloat16), w1_ref[...],
                preferred_element_type=jnp.float32)
        + b1_ref[...], 0.0)
    hw = jnp.dot(h.astype(jnp.bfloat16), w2_ref[...],
                 preferred_element_type=jnp.float32)
    o_ref[...] = hw * dinv


def _fin_body(dp_ref, q0_ref, q1_ref, hwp_ref, b2_ref, o_ref):
    dinv = _dinv_of(dp_ref)
    o_ref[...] = (q0_ref[...] + q1_ref[...] + hwp_ref[...]) * dinv + b2_ref[...]


def _row_spec(width):
    return pl.BlockSpec((BLK, width), lambda i: (i, 0))


def _full_spec(shape):
    return pl.BlockSpec(shape, lambda i: (0,) * len(shape))


_scale_call = pl.pallas_call(
    _scale_body,
    grid=(GRID,),
    in_specs=[_row_spec(2), _row_spec(IN_DIM)],
    out_specs=_row_spec(IN_DIM),
    out_shape=jax.ShapeDtypeStruct((NPAD, IN_DIM), jnp.float32),
)

_mm_call = pl.pallas_call(
    _mm_body,
    grid=(GRID,),
    in_specs=[_row_spec(2), _row_spec(128), _row_spec(128), _row_spec(IN_DIM),
              _full_spec((IN_DIM, HID)), _full_spec((1, HID)),
              _full_spec((HID, Z_DIM))],
    out_specs=_row_spec(Z_DIM),
    out_shape=jax.ShapeDtypeStruct((NPAD, Z_DIM), jnp.float32),
)

_fin_call = pl.pallas_call(
    _fin_body,
    grid=(GRID,),
    in_specs=[_row_spec(2), _row_spec(Z_DIM), _row_spec(Z_DIM),
              _row_spec(Z_DIM), _full_spec((1, Z_DIM))],
    out_specs=_row_spec(Z_DIM),
    out_shape=jax.ShapeDtypeStruct((NPAD, Z_DIM), jnp.float32),
)


def kernel(x, ei, W1, b1, W2, b2):
    src = ei[0].astype(jnp.int32)
    dst = ei[1].astype(jnp.int32)
    pad_rows = N_NODES + (jnp.arange(EPAD, dtype=jnp.int32) % (NPAD - N_NODES))
    src_p = pad_rows.at[:N_EDGES].set(src)
    dst_p = pad_rows.at[:N_EDGES].set(dst)
    packed = ((src_p << IDX_BITS) | dst_p).reshape(NW, NCHUNK, CH)
    dst_p = dst_p.reshape(NW, NCHUNK, CH)
    x_pad = jnp.zeros((NPAD, IN_DIM), jnp.float32).at[:N_NODES].set(x)
    zeros1 = jnp.zeros((NPAD,), jnp.float32)
    zeros2 = jnp.zeros((NPAD, 128), jnp.float32)

    ones_ch = jnp.ones((NCHUNK * CH,), jnp.float32)
    degp = _deg_kernel(dst_p.reshape(NW, NCHUNK * CH), ones_ch,
                       zeros1)
    degp_t = degp.T
    xp = _scale_call(degp_t, x_pad)
    p = _agg_kernel(xp, packed, zeros2)
    hwp = _mm_call(degp_t, p[0], p[1], xp, W1.astype(jnp.bfloat16),
                   b1.reshape(1, HID), W2.astype(jnp.bfloat16))
    q = _agg_kernel(hwp, packed, zeros2)
    z = _fin_call(degp_t, q[0], q[1], hwp,
                  b2.reshape(1, Z_DIM))
    return z[:N_NODES]


# speedup vs baseline: 1.0377x; 1.0377x over previous
import functools

import jax
import jax.numpy as jnp
from jax import lax
from jax.experimental import pallas as pl
from jax.experimental.pallas import tpu as pltpu
from jax.experimental.pallas import tpu_sc as plsc

N_NODES = 10000
N_EDGES = 320000
IN_DIM = 128
HID = 256
Z_DIM = 128

NPAD = 10240
NCORE = 2
NSUB = 16
NW = NCORE * NSUB
CH = 64
NBUF = 3
IDX_BITS = 14
NCHUNK = (N_EDGES + NW * CH - 1) // (NW * CH)
EPAD = NW * NCHUNK * CH

ROWS_PER_SUB = NPAD // NSUB
BLK = 512
GRID = NPAD // BLK

_mesh = plsc.VectorSubcoreMesh(core_axis_name="c", subcore_axis_name="s")


@functools.partial(
    pl.kernel,
    out_type=jax.ShapeDtypeStruct((NCORE, NPAD), jnp.float32),
    mesh=_mesh,
    scratch_types=[
        pltpu.VMEM((NCHUNK * CH,), jnp.int32),
        pltpu.VMEM((NCHUNK * CH,), jnp.float32),
        pltpu.VMEM_SHARED((NPAD,), jnp.float32),
    ],
)
def _deg_kernel(dst_hbm, ones_hbm, zeros1_hbm, degp_hbm, dst_v, ones_v, deg_sh):
    c = lax.axis_index("c")
    s = lax.axis_index("s")
    wid = c * NSUB + s
    pltpu.sync_copy(dst_hbm.at[wid], dst_v)
    pltpu.sync_copy(ones_hbm, ones_v)
    pltpu.sync_copy(zeros1_hbm.at[pl.ds(s * ROWS_PER_SUB, ROWS_PER_SUB)],
                    deg_sh.at[pl.ds(s * ROWS_PER_SUB, ROWS_PER_SUB)])
    plsc.subcore_barrier()

    pltpu.sync_copy(ones_v, deg_sh.at[dst_v], add=True)
    plsc.subcore_barrier()

    @pl.when(s == 0)
    def _():
        pltpu.sync_copy(deg_sh, degp_hbm.at[c])


@functools.partial(
    pl.kernel,
    out_type=jax.ShapeDtypeStruct((NCORE, NPAD, 128), jnp.float32),
    mesh=_mesh,
    scratch_types=[
        pltpu.VMEM((NCHUNK, CH), jnp.int32),
        pltpu.VMEM((NBUF, CH), jnp.int32),
        pltpu.VMEM((NBUF, CH), jnp.int32),
        pltpu.VMEM((CH, 128), jnp.float32),
        pltpu.VMEM((CH, 128), jnp.float32),
        pltpu.VMEM((CH, 128), jnp.float32),
        pltpu.VMEM_SHARED((NPAD, 128), jnp.float32),
        pltpu.SemaphoreType.DMA,
        pltpu.SemaphoreType.DMA,
        pltpu.SemaphoreType.DMA,
    ],
)
def _agg_kernel(xp_hbm, packed_hbm, zeros2_hbm, out_hbm,
                packed_v, sidx, didx, rows0, rows1, rows2, acc_sh,
                sem0, sem1, sem2):
    c = lax.axis_index("c")
    s = lax.axis_index("s")
    wid = c * NSUB + s
    pltpu.sync_copy(packed_hbm.at[wid], packed_v)
    pltpu.sync_copy(zeros2_hbm.at[pl.ds(s * ROWS_PER_SUB, ROWS_PER_SUB)],
                    acc_sh.at[pl.ds(s * ROWS_PER_SUB, ROWS_PER_SUB)])
    plsc.subcore_barrier()

    def unpack(j, b):
        for k in range(CH // 16):
            v = packed_v[j, pl.ds(k * 16, 16)]
            sidx[b, pl.ds(k * 16, 16)] = lax.shift_right_logical(v, IDX_BITS)
            didx[b, pl.ds(k * 16, 16)] = lax.bitwise_and(v, (1 << IDX_BITS) - 1)

    bufs = (rows0, rows1, rows2)
    sems = (sem0, sem1, sem2)
    for b in range(NBUF):
        unpack(b, b)
        pltpu.async_copy(xp_hbm.at[sidx.at[b]], bufs[b], sems[b])

    def body(g, carry):
        for b in range(NBUF):
            j = g * NBUF + b
            pltpu.make_async_copy(xp_hbm.at[pl.ds(0, CH)], bufs[b],
                                  sems[b]).wait()
            pltpu.sync_copy(bufs[b], acc_sh.at[didx.at[b]], add=True)

            @pl.when(j + NBUF < NCHUNK)
            def _():
                unpack(j + NBUF, b)
                pltpu.async_copy(xp_hbm.at[sidx.at[b]], bufs[b], sems[b])

        return carry

    lax.fori_loop(0, NCHUNK // NBUF, body, 0)
    for r in range(NCHUNK - NBUF * (NCHUNK // NBUF)):
        j = NBUF * (NCHUNK // NBUF) + r
        pltpu.make_async_copy(xp_hbm.at[pl.ds(0, CH)], bufs[j % NBUF],
                              sems[j % NBUF]).wait()
        pltpu.sync_copy(bufs[j % NBUF], acc_sh.at[didx.at[j % NBUF]], add=True)
    plsc.subcore_barrier()
    pltpu.sync_copy(acc_sh.at[pl.ds(s * ROWS_PER_SUB, ROWS_PER_SUB)],
                    out_hbm.at[c, pl.ds(s * ROWS_PER_SUB, ROWS_PER_SUB)])


def _dinv_of(dp_ref):
    return lax.rsqrt(dp_ref[:, 0:1] + dp_ref[:, 1:2] + 1.0)


def _scale_body(dp_ref, x_ref, o_ref):
    o_ref[...] = x_ref[...] * _dinv_of(dp_ref)


def _mm_body(dp_ref, p0_ref, p1_ref, xp_ref, w1_ref, b1_ref, w2_ref, o_ref):
    dinv = _dinv_of(dp_ref)
    agg1 = (p0_ref[...] + p1_ref[...] + xp_ref[...]) * dinv
    h = jnp.maximum(
        jnp.dot(agg1, w1_ref[...], preferred_element_type=jnp.float32)
        + b1_ref[...], 0.0)
    hw = jnp.dot(h, w2_ref[...], preferred_element_type=jnp.float32)
    o_ref[...] = hw * dinv


def _fin_body(dp_ref, q0_ref, q1_ref, hwp_ref, b2_ref, o_ref):
    dinv = _dinv_of(dp_ref)
    o_ref[...] = (q0_ref[...] + q1_ref[...] + hwp_ref[...]) * dinv + b2_ref[...]


def _row_spec(width):
    return pl.BlockSpec((BLK, width), lambda i: (i, 0))


def _full_spec(shape):
    return pl.BlockSpec(shape, lambda i: (0,) * len(shape))


_scale_call = pl.pallas_call(
    _scale_body,
    grid=(GRID,),
    in_specs=[_row_spec(2), _row_spec(IN_DIM)],
    out_specs=_row_spec(IN_DIM),
    out_shape=jax.ShapeDtypeStruct((NPAD, IN_DIM), jnp.float32),
)

_mm_call = pl.pallas_call(
    _mm_body,
    grid=(GRID,),
    in_specs=[_row_spec(2), _row_spec(128), _row_spec(128), _row_spec(IN_DIM),
              _full_spec((IN_DIM, HID)), _full_spec((1, HID)),
              _full_spec((HID, Z_DIM))],
    out_specs=_row_spec(Z_DIM),
    out_shape=jax.ShapeDtypeStruct((NPAD, Z_DIM), jnp.float32),
)

_fin_call = pl.pallas_call(
    _fin_body,
    grid=(GRID,),
    in_specs=[_row_spec(2), _row_spec(Z_DIM), _row_spec(Z_DIM),
              _row_spec(Z_DIM), _full_spec((1, Z_DIM))],
    out_specs=_row_spec(Z_DIM),
    out_shape=jax.ShapeDtypeStruct((NPAD, Z_DIM), jnp.float32),
)


def kernel(x, ei, W1, b1, W2, b2):
    src = ei[0].astype(jnp.int32)
    dst = ei[1].astype(jnp.int32)
    pad_rows = N_NODES + (jnp.arange(EPAD, dtype=jnp.int32) % (NPAD - N_NODES))
    src_p = pad_rows.at[:N_EDGES].set(src)
    dst_p = pad_rows.at[:N_EDGES].set(dst)
    packed = ((src_p << IDX_BITS) | dst_p).reshape(NW, NCHUNK, CH)
    dst_p = dst_p.reshape(NW, NCHUNK, CH)
    x_pad = jnp.zeros((NPAD, IN_DIM), jnp.float32).at[:N_NODES].set(x)
    zeros1 = jnp.zeros((NPAD,), jnp.float32)
    zeros2 = jnp.zeros((NPAD, 128), jnp.float32)

    ones_ch = jnp.ones((NCHUNK * CH,), jnp.float32)
    degp = _deg_kernel(dst_p.reshape(NW, NCHUNK * CH), ones_ch,
                       zeros1)
    degp_t = degp.T
    xp = _scale_call(degp_t, x_pad)
    p = _agg_kernel(xp, packed, zeros2)
    hwp = _mm_call(degp_t, p[0], p[1], xp, W1,
                   b1.reshape(1, HID), W2)
    q = _agg_kernel(hwp, packed, zeros2)
    z = _fin_call(degp_t, q[0], q[1], hwp,
                  b2.reshape(1, Z_DIM))
    return z[:N_NODES]


# speedup vs baseline: 1.1044x; 1.0643x over previous
import functools

import jax
import jax.numpy as jnp
from jax import lax
from jax.experimental import pallas as pl
from jax.experimental.pallas import tpu as pltpu
from jax.experimental.pallas import tpu_sc as plsc

N_NODES = 10000
N_EDGES = 320000
IN_DIM = 128
HID = 256
Z_DIM = 128

NPAD = 10240
NCORE = 2
NSUB = 16
NW = NCORE * NSUB
CH = 80
NBUF = 3
IDX_BITS = 14
NCHUNK = (N_EDGES + NW * CH - 1) // (NW * CH)
EPAD = NW * NCHUNK * CH

ROWS_PER_SUB = NPAD // NSUB
BLK = 512
GRID = NPAD // BLK

_mesh = plsc.VectorSubcoreMesh(core_axis_name="c", subcore_axis_name="s")


@functools.partial(
    pl.kernel,
    out_type=jax.ShapeDtypeStruct((NCORE, NPAD), jnp.float32),
    mesh=_mesh,
    scratch_types=[
        pltpu.VMEM((NCHUNK * CH,), jnp.int32),
        pltpu.VMEM((NCHUNK * CH,), jnp.float32),
        pltpu.VMEM_SHARED((NPAD,), jnp.float32),
    ],
)
def _deg_kernel(dst_hbm, ones_hbm, zeros1_hbm, degp_hbm, dst_v, ones_v, deg_sh):
    c = lax.axis_index("c")
    s = lax.axis_index("s")
    wid = c * NSUB + s
    pltpu.sync_copy(dst_hbm.at[wid], dst_v)
    pltpu.sync_copy(ones_hbm, ones_v)
    pltpu.sync_copy(zeros1_hbm.at[pl.ds(s * ROWS_PER_SUB, ROWS_PER_SUB)],
                    deg_sh.at[pl.ds(s * ROWS_PER_SUB, ROWS_PER_SUB)])
    plsc.subcore_barrier()

    pltpu.sync_copy(ones_v, deg_sh.at[dst_v], add=True)
    plsc.subcore_barrier()

    @pl.when(s == 0)
    def _():
        pltpu.sync_copy(deg_sh, degp_hbm.at[c])


@functools.partial(
    pl.kernel,
    out_type=jax.ShapeDtypeStruct((NCORE, NPAD, 128), jnp.float32),
    mesh=_mesh,
    scratch_types=[
        pltpu.VMEM((NCHUNK, CH), jnp.int32),
        pltpu.VMEM((NBUF, CH), jnp.int32),
        pltpu.VMEM((NBUF, CH), jnp.int32),
        pltpu.VMEM((CH, 128), jnp.float32),
        pltpu.VMEM((CH, 128), jnp.float32),
        pltpu.VMEM((CH, 128), jnp.float32),
        pltpu.VMEM_SHARED((NPAD, 128), jnp.float32),
        pltpu.SemaphoreType.DMA,
        pltpu.SemaphoreType.DMA,
        pltpu.SemaphoreType.DMA,
    ],
)
def _agg_kernel(xp_hbm, packed_hbm, zeros2_hbm, out_hbm,
                packed_v, sidx, didx, rows0, rows1, rows2, acc_sh,
                sem0, sem1, sem2):
    c = lax.axis_index("c")
    s = lax.axis_index("s")
    wid = c * NSUB + s
    pltpu.sync_copy(packed_hbm.at[wid], packed_v)
    pltpu.sync_copy(zeros2_hbm.at[pl.ds(s * ROWS_PER_SUB, ROWS_PER_SUB)],
                    acc_sh.at[pl.ds(s * ROWS_PER_SUB, ROWS_PER_SUB)])
    plsc.subcore_barrier()

    def unpack(j, b):
        for k in range(CH // 16):
            v = packed_v[j, pl.ds(k * 16, 16)]
            sidx[b, pl.ds(k * 16, 16)] = lax.shift_right_logical(v, IDX_BITS)
            didx[b, pl.ds(k * 16, 16)] = lax.bitwise_and(v, (1 << IDX_BITS) - 1)

    bufs = (rows0, rows1, rows2)
    sems = (sem0, sem1, sem2)
    for b in range(NBUF):
        unpack(b, b)
        pltpu.async_copy(xp_hbm.at[sidx.at[b]], bufs[b], sems[b])

    def body(g, carry):
        for b in range(NBUF):
            j = g * NBUF + b
            pltpu.make_async_copy(xp_hbm.at[pl.ds(0, CH)], bufs[b],
                                  sems[b]).wait()
            pltpu.sync_copy(bufs[b], acc_sh.at[didx.at[b]], add=True)

            @pl.when(j + NBUF < NCHUNK)
            def _():
                unpack(j + NBUF, b)
                pltpu.async_copy(xp_hbm.at[sidx.at[b]], bufs[b], sems[b])

        return carry

    lax.fori_loop(0, NCHUNK // NBUF, body, 0)
    for r in range(NCHUNK - NBUF * (NCHUNK // NBUF)):
        j = NBUF * (NCHUNK // NBUF) + r
        pltpu.make_async_copy(xp_hbm.at[pl.ds(0, CH)], bufs[j % NBUF],
                              sems[j % NBUF]).wait()
        pltpu.sync_copy(bufs[j % NBUF], acc_sh.at[didx.at[j % NBUF]], add=True)
    plsc.subcore_barrier()
    pltpu.sync_copy(acc_sh.at[pl.ds(s * ROWS_PER_SUB, ROWS_PER_SUB)],
                    out_hbm.at[c, pl.ds(s * ROWS_PER_SUB, ROWS_PER_SUB)])


def _dinv_of(dp_ref):
    return lax.rsqrt(dp_ref[:, 0:1] + dp_ref[:, 1:2] + 1.0)


def _scale_body(dp_ref, x_ref, o_ref):
    o_ref[...] = x_ref[...] * _dinv_of(dp_ref)


def _mm_body(dp_ref, p0_ref, p1_ref, xp_ref, w1_ref, b1_ref, w2_ref, o_ref):
    dinv = _dinv_of(dp_ref)
    agg1 = (p0_ref[...] + p1_ref[...] + xp_ref[...]) * dinv
    h = jnp.maximum(
        jnp.dot(agg1, w1_ref[...], preferred_element_type=jnp.float32)
        + b1_ref[...], 0.0)
    hw = jnp.dot(h, w2_ref[...], preferred_element_type=jnp.float32)
    o_ref[...] = hw * dinv


def _fin_body(dp_ref, q0_ref, q1_ref, hwp_ref, b2_ref, o_ref):
    dinv = _dinv_of(dp_ref)
    o_ref[...] = (q0_ref[...] + q1_ref[...] + hwp_ref[...]) * dinv + b2_ref[...]


def _row_spec(width):
    return pl.BlockSpec((BLK, width), lambda i: (i, 0))


def _full_spec(shape):
    return pl.BlockSpec(shape, lambda i: (0,) * len(shape))


_scale_call = pl.pallas_call(
    _scale_body,
    grid=(GRID,),
    in_specs=[_row_spec(2), _row_spec(IN_DIM)],
    out_specs=_row_spec(IN_DIM),
    out_shape=jax.ShapeDtypeStruct((NPAD, IN_DIM), jnp.float32),
)

_mm_call = pl.pallas_call(
    _mm_body,
    grid=(GRID,),
    in_specs=[_row_spec(2), _row_spec(128), _row_spec(128), _row_spec(IN_DIM),
              _full_spec((IN_DIM, HID)), _full_spec((1, HID)),
              _full_spec((HID, Z_DIM))],
    out_specs=_row_spec(Z_DIM),
    out_shape=jax.ShapeDtypeStruct((NPAD, Z_DIM), jnp.float32),
)

_fin_call = pl.pallas_call(
    _fin_body,
    grid=(GRID,),
    in_specs=[_row_spec(2), _row_spec(Z_DIM), _row_spec(Z_DIM),
              _row_spec(Z_DIM), _full_spec((1, Z_DIM))],
    out_specs=_row_spec(Z_DIM),
    out_shape=jax.ShapeDtypeStruct((NPAD, Z_DIM), jnp.float32),
)


def kernel(x, ei, W1, b1, W2, b2):
    src = ei[0].astype(jnp.int32)
    dst = ei[1].astype(jnp.int32)
    pad_rows = N_NODES + (jnp.arange(EPAD, dtype=jnp.int32) % (NPAD - N_NODES))
    src_p = pad_rows.at[:N_EDGES].set(src)
    dst_p = pad_rows.at[:N_EDGES].set(dst)
    packed = ((src_p << IDX_BITS) | dst_p).reshape(NW, NCHUNK, CH)
    dst_p = dst_p.reshape(NW, NCHUNK, CH)
    x_pad = jnp.zeros((NPAD, IN_DIM), jnp.float32).at[:N_NODES].set(x)
    zeros1 = jnp.zeros((NPAD,), jnp.float32)
    zeros2 = jnp.zeros((NPAD, 128), jnp.float32)

    ones_ch = jnp.ones((NCHUNK * CH,), jnp.float32)
    degp = _deg_kernel(dst_p.reshape(NW, NCHUNK * CH), ones_ch,
                       zeros1)
    degp_t = degp.T
    xp = _scale_call(degp_t, x_pad)
    p = _agg_kernel(xp, packed, zeros2)
    hwp = _mm_call(degp_t, p[0], p[1], xp, W1,
                   b1.reshape(1, HID), W2)
    q = _agg_kernel(hwp, packed, zeros2)
    z = _fin_call(degp_t, q[0], q[1], hwp,
                  b2.reshape(1, Z_DIM))
    return z[:N_NODES]
